# baseline (speedup 1.0000x reference)
import functools

import jax
import jax.numpy as jnp
from jax import lax
from jax.experimental import pallas as pl
from jax.experimental.pallas import tpu as pltpu
from jax.experimental.pallas import tpu_sc as plsc


def _make_sc_embed(B, D, NC, NS, L):
    NW = NC * NS
    b_per_w = B // NW
    CH = 128
    n_ch = b_per_w // CH

    mesh = plsc.VectorSubcoreMesh(core_axis_name="c", subcore_axis_name="s")

    @functools.partial(
        pl.kernel,
        mesh=mesh,
        compiler_params=pltpu.CompilerParams(use_tc_tiling_on_sc=False),
        out_type=jax.ShapeDtypeStruct((B, D), jnp.float32),
        scratch_types=[
            pltpu.VMEM((n_ch, CH), jnp.int32),
            pltpu.VMEM((n_ch, CH, D), jnp.float32),
            pltpu.VMEM((L,), jnp.float32),
            pltpu.SemaphoreType.DMA((n_ch,)),
            pltpu.SemaphoreType.DMA,
        ],
    )
    def sc_embed(tok_hbm, scale_hbm, table_hbm, out_hbm,
                 idx_v, rows_v, scale_v, gsems, ssem):
        wid = lax.axis_index("s") * NC + lax.axis_index("c")
        base = wid * b_per_w
        pltpu.sync_copy(tok_hbm.at[wid], idx_v)
        pltpu.sync_copy(scale_hbm, scale_v)
        gathers = []
        for c in range(n_ch):
            gathers.append(pltpu.async_copy(
                table_hbm.at[idx_v.at[c]], rows_v.at[c], gsems.at[c]))
        s = scale_v[...]
        stores = []
        for c in range(n_ch):
            gathers[c].wait()

            def mul_row(i, carry, c=c):
                for j in range(D // L):
                    sl = (i, pl.ds(j * L, L))
                    rows_v.at[c][sl] = rows_v.at[c][sl] * s
                return carry

            lax.fori_loop(0, CH, mul_row, 0)
            stores.append(pltpu.async_copy(
                rows_v.at[c], out_hbm.at[pl.ds(base + c * CH, CH)], ssem))
        for st in stores:
            st.wait()

    return sc_embed


def kernel(token_ids, embed_weight, scale):
    B0, B1 = token_ids.shape
    V, D = embed_weight.shape
    B = B0 * B1
    info = plsc.get_sparse_core_info()
    NC, NS, L = info.num_cores, info.num_subcores, info.num_lanes
    NW = NC * NS
    b_per_w = B // NW
    CH = 128
    n_ch = b_per_w // CH

    tok = token_ids.reshape(NW, n_ch, CH).astype(jnp.int32)
    scale16 = jnp.broadcast_to(
        scale.astype(jnp.float32).reshape(1), (L,))
    out = _make_sc_embed(B, D, NC, NS, L)(tok, scale16, embed_weight)
    return out.reshape(B0, B1, D)
